# baseline (speedup 1.0000x reference)
import jax
import jax.numpy as jnp
from jax import lax
from jax.experimental import pallas as pl
from jax.experimental.pallas import tpu as pltpu
from jax.experimental.pallas import tpu_sc as plsc

N_ENV = 512
BUF = 1024
N_OBS = 64
N_ACT = 16
BATCH = 256

NC = 2
NS = 16
NW = NC * NS
L = 16
EPW = N_ENV // NW
NCHUNK = 2
CH = BATCH // NCHUNK
NG = BATCH // L


def _worker_id():
    return lax.axis_index("s") * NC + lax.axis_index("c")


def _body(obs_flat, nobs_flat, act_flat, rew_buf, don_buf, ter_buf, tou_buf,
          obs_new, nobs_new, act_new, rew_new, don_new, ter_new, tou_new,
          idx3, p_arr,
          obs_out, nobs_out, act_out, rew_out, don_out, ter_out, tou_out,
          ens_out,
          idx_v, gidx_a, gidx_b,
          obs_rows_a, obs_rows_b, nobs_rows_a, nobs_rows_b,
          act_rows_a, act_rows_b,
          rew_row, don_row, ter_row, tou_row,
          rew_so, don_so, ter_so, tou_so, ens_so,
          obs_ne, nobs_ne, act_ne,
          rew16, don16, ter16, tou16, p_v,
          sem_g, sem_s, sem_o):
    w = _worker_id()
    base_env = w * EPW

    pltpu.sync_copy(p_arr, p_v)
    pltpu.sync_copy(rew_new.at[pl.ds(base_env, EPW)], rew16.at[pl.ds(0, EPW)])
    pltpu.sync_copy(don_new.at[pl.ds(base_env, EPW)], don16.at[pl.ds(0, EPW)])
    pltpu.sync_copy(ter_new.at[pl.ds(base_env, EPW)], ter16.at[pl.ds(0, EPW)])
    pltpu.sync_copy(tou_new.at[pl.ds(base_env, EPW)], tou16.at[pl.ds(0, EPW)])
    pv = p_v[...]
    ones16 = jnp.ones((L,), jnp.int32)
    for g in range(NG):
        ens_so[pl.ds(g * L, L)] = ones16

    def env_body(j, carry):
        e = base_env + j
        pltpu.sync_copy(idx3.at[e], idx_v)

        ebase = e * BUF
        hit_acc = jnp.zeros((L,), jnp.bool_)
        gidx = (gidx_a, gidx_b)
        for g in range(NG):
            iv = idx_v[g // 8, pl.ds((g % 8) * L, L)]
            gidx[g // 8][pl.ds((g % 8) * L, L)] = iv + ebase
            hit_acc = jnp.logical_or(hit_acc, iv == pv)
        anyhit = plsc.all_reduce_population_count(hit_acc)[0] > 0

        @pl.when(j > 0)
        def _drain_prev():
            ob_p = e * BATCH
            for c in range(NCHUNK):
                pltpu.make_async_copy(
                    obs_rows_a, obs_out.at[pl.ds(ob_p, CH)], sem_o).wait()
                pltpu.make_async_copy(
                    nobs_rows_a, nobs_out.at[pl.ds(ob_p, CH)], sem_o).wait()
                pltpu.make_async_copy(
                    act_rows_a, act_out.at[pl.ds(ob_p, CH)], sem_o).wait()
            pltpu.make_async_copy(
                rew_so, rew_out.at[pl.ds(ob_p, BATCH)], sem_o).wait()
            pltpu.make_async_copy(
                don_so, don_out.at[pl.ds(ob_p, BATCH)], sem_o).wait()
            pltpu.make_async_copy(
                ter_so, ter_out.at[pl.ds(ob_p, BATCH)], sem_o).wait()
            pltpu.make_async_copy(
                tou_so, tou_out.at[pl.ds(ob_p, BATCH)], sem_o).wait()
            pltpu.make_async_copy(
                ens_so, ens_out.at[pl.ds(ob_p, BATCH)], sem_o).wait()

        obs_rows = (obs_rows_a, obs_rows_b)
        nobs_rows = (nobs_rows_a, nobs_rows_b)
        act_rows = (act_rows_a, act_rows_b)
        g_copies = []
        for c in range(NCHUNK):
            g_copies.append(pltpu.async_copy(
                obs_flat.at[gidx[c]], obs_rows[c], sem_g))
            g_copies.append(pltpu.async_copy(
                nobs_flat.at[gidx[c]], nobs_rows[c], sem_g))
            g_copies.append(pltpu.async_copy(
                act_flat.at[gidx[c]], act_rows[c], sem_g))
        s_copies = [
            pltpu.async_copy(rew_buf.at[e], rew_row, sem_s),
            pltpu.async_copy(don_buf.at[e], don_row, sem_s),
            pltpu.async_copy(ter_buf.at[e], ter_row, sem_s),
            pltpu.async_copy(tou_buf.at[e], tou_row, sem_s),
        ]

        @pl.when(anyhit)
        def _load_new():
            pltpu.sync_copy(obs_new.at[e], obs_ne)
            pltpu.sync_copy(nobs_new.at[e], nobs_ne)
            pltpu.sync_copy(act_new.at[e], act_ne)

        rew_e = jnp.full((L,), rew16[pl.ds(j, L)][0])
        don_e = jnp.full((L,), don16[pl.ds(j, L)][0])
        ter_e = jnp.full((L,), ter16[pl.ds(j, L)][0])
        tou_e = jnp.full((L,), tou16[pl.ds(j, L)][0])

        for cp in s_copies:
            cp.wait()
        for g in range(NG):
            iv = idx_v[g // 8, pl.ds((g % 8) * L, L)]
            m = iv == pv
            rew_so[pl.ds(g * L, L)] = jnp.where(
                m, rew_e, plsc.load_gather(rew_row, [iv]))
            don_so[pl.ds(g * L, L)] = jnp.where(
                m, don_e, plsc.load_gather(don_row, [iv]))
            ter_so[pl.ds(g * L, L)] = jnp.where(
                m, ter_e, plsc.load_gather(ter_row, [iv]))
            tou_so[pl.ds(g * L, L)] = jnp.where(
                m, tou_e, plsc.load_gather(tou_row, [iv]))

        for cp in g_copies:
            cp.wait()

        @pl.when(anyhit)
        def _fix():
            onew = [obs_ne[pl.ds(k * L, L)] for k in range(N_OBS // L)]
            nnew = [nobs_ne[pl.ds(k * L, L)] for k in range(N_OBS // L)]
            anew = act_ne[...]
            for g in range(NG):
                iv = idx_v[g // 8, pl.ds((g % 8) * L, L)]
                m = iv == pv
                mi = jnp.where(m, 1, 0).astype(jnp.int32)

                @pl.when(plsc.all_reduce_population_count(m)[0] > 0)
                def _fix_group(g=g, mi=mi):
                    ck = g // 8
                    for lane in range(L):
                        @pl.when(mi[lane] != 0)
                        def _fix_lane(g=g, lane=lane, ck=ck):
                            b = (g % 8) * L + lane
                            for k in range(N_OBS // L):
                                obs_rows[ck][b, pl.ds(k * L, L)] = onew[k]
                                nobs_rows[ck][b, pl.ds(k * L, L)] = nnew[k]
                            act_rows[ck][b, :] = anew

        ob = e * BATCH
        o_copies = []
        for c in range(NCHUNK):
            o_copies.append(pltpu.async_copy(
                obs_rows[c], obs_out.at[pl.ds(ob + c * CH, CH)], sem_o))
            o_copies.append(pltpu.async_copy(
                nobs_rows[c], nobs_out.at[pl.ds(ob + c * CH, CH)], sem_o))
            o_copies.append(pltpu.async_copy(
                act_rows[c], act_out.at[pl.ds(ob + c * CH, CH)], sem_o))
        o_copies.append(pltpu.async_copy(
            rew_so, rew_out.at[pl.ds(ob, BATCH)], sem_o))
        o_copies.append(pltpu.async_copy(
            don_so, don_out.at[pl.ds(ob, BATCH)], sem_o))
        o_copies.append(pltpu.async_copy(
            ter_so, ter_out.at[pl.ds(ob, BATCH)], sem_o))
        o_copies.append(pltpu.async_copy(
            tou_so, tou_out.at[pl.ds(ob, BATCH)], sem_o))
        o_copies.append(pltpu.async_copy(
            ens_so, ens_out.at[pl.ds(ob, BATCH)], sem_o))
        return carry

    lax.fori_loop(0, EPW, env_body, 0)

    e_last = base_env + EPW - 1
    ob_l = e_last * BATCH
    for c in range(NCHUNK):
        pltpu.make_async_copy(
            obs_rows_a, obs_out.at[pl.ds(ob_l, CH)], sem_o).wait()
        pltpu.make_async_copy(
            nobs_rows_a, nobs_out.at[pl.ds(ob_l, CH)], sem_o).wait()
        pltpu.make_async_copy(
            act_rows_a, act_out.at[pl.ds(ob_l, CH)], sem_o).wait()
    pltpu.make_async_copy(
        rew_so, rew_out.at[pl.ds(ob_l, BATCH)], sem_o).wait()
    pltpu.make_async_copy(
        don_so, don_out.at[pl.ds(ob_l, BATCH)], sem_o).wait()
    pltpu.make_async_copy(
        ter_so, ter_out.at[pl.ds(ob_l, BATCH)], sem_o).wait()
    pltpu.make_async_copy(
        tou_so, tou_out.at[pl.ds(ob_l, BATCH)], sem_o).wait()
    pltpu.make_async_copy(
        ens_so, ens_out.at[pl.ds(ob_l, BATCH)], sem_o).wait()


_OUT_TYPE = (
    jax.ShapeDtypeStruct((N_ENV * BATCH, N_OBS), jnp.float32),
    jax.ShapeDtypeStruct((N_ENV * BATCH, N_OBS), jnp.float32),
    jax.ShapeDtypeStruct((N_ENV * BATCH, N_ACT), jnp.float32),
    jax.ShapeDtypeStruct((N_ENV * BATCH,), jnp.float32),
    jax.ShapeDtypeStruct((N_ENV * BATCH,), jnp.int32),
    jax.ShapeDtypeStruct((N_ENV * BATCH,), jnp.int32),
    jax.ShapeDtypeStruct((N_ENV * BATCH,), jnp.int32),
    jax.ShapeDtypeStruct((N_ENV * BATCH,), jnp.int32),
)

_SCRATCH = [
    pltpu.VMEM((NCHUNK, CH), jnp.int32),
    pltpu.VMEM((CH,), jnp.int32),
    pltpu.VMEM((CH,), jnp.int32),
    pltpu.VMEM((CH, N_OBS), jnp.float32),
    pltpu.VMEM((CH, N_OBS), jnp.float32),
    pltpu.VMEM((CH, N_OBS), jnp.float32),
    pltpu.VMEM((CH, N_OBS), jnp.float32),
    pltpu.VMEM((CH, N_ACT), jnp.float32),
    pltpu.VMEM((CH, N_ACT), jnp.float32),
    pltpu.VMEM((BUF,), jnp.float32),
    pltpu.VMEM((BUF,), jnp.int32),
    pltpu.VMEM((BUF,), jnp.int32),
    pltpu.VMEM((BUF,), jnp.int32),
    pltpu.VMEM((BATCH,), jnp.float32),
    pltpu.VMEM((BATCH,), jnp.int32),
    pltpu.VMEM((BATCH,), jnp.int32),
    pltpu.VMEM((BATCH,), jnp.int32),
    pltpu.VMEM((BATCH,), jnp.int32),
    pltpu.VMEM((N_OBS,), jnp.float32),
    pltpu.VMEM((N_OBS,), jnp.float32),
    pltpu.VMEM((N_ACT,), jnp.float32),
    pltpu.VMEM((EPW + L,), jnp.float32),
    pltpu.VMEM((EPW + L,), jnp.int32),
    pltpu.VMEM((EPW + L,), jnp.int32),
    pltpu.VMEM((EPW + L,), jnp.int32),
    pltpu.VMEM((L,), jnp.int32),
    pltpu.SemaphoreType.DMA,
    pltpu.SemaphoreType.DMA,
    pltpu.SemaphoreType.DMA,
]

_sc_call = pl.kernel(
    _body,
    out_type=_OUT_TYPE,
    mesh=plsc.VectorSubcoreMesh(core_axis_name="c", subcore_axis_name="s",
                                num_cores=NC, num_subcores=NS),
    scratch_types=_SCRATCH,
    compiler_params=pltpu.CompilerParams(needs_layout_passes=False,
                                         use_tc_tiling_on_sc=False),
)


def kernel(observations_buf, next_observations_buf, actions_buf, rewards_buf,
           dones_buf, terminations_buf, time_outs_buf,
           obs, actions_in, rewards_in, next_obs,
           dones_in, terminations_in, time_outs_in,
           indices, ptr):
    p = jnp.asarray(ptr, jnp.int32) % BUF
    p_arr = jnp.full((L,), p, jnp.int32)
    obs_flat = observations_buf.reshape(N_ENV * BUF, N_OBS)
    nobs_flat = next_observations_buf.reshape(N_ENV * BUF, N_OBS)
    act_flat = actions_buf.reshape(N_ENV * BUF, N_ACT)
    idx3 = indices.reshape(N_ENV, NCHUNK, CH)
    return _sc_call(
        obs_flat, nobs_flat, act_flat, rewards_buf, dones_buf,
        terminations_buf, time_outs_buf,
        obs, next_obs, actions_in, rewards_in,
        dones_in, terminations_in, time_outs_in,
        idx3, p_arr)
